# baseline (speedup 1.0000x reference)
import jax
import jax.numpy as jnp
from jax import lax
from jax.experimental import pallas as pl
from jax.experimental.pallas import tpu as pltpu
from jax.experimental.pallas import tpu_sc as plsc

VOCAB = 1000000
D = 64
B = 4096
L = 200
NW = 32
BPW = B // NW
NGRP = BPW // 16


_NS = 4
_BLK = 8192
_STEP = _NS * _BLK
_TCG = 15
_VPAD = _TCG * _STEP


def _tc_dot_body(*refs):
    w_ref, o_ref = refs[_NS], refs[_NS + 1]
    for s in range(_NS):
        y = jnp.dot(refs[s][...], w_ref[...], preferred_element_type=jnp.float32)
        o_ref[:, s * _BLK:(s + 1) * _BLK] = y.T


_TOTB = -(-VOCAB // _BLK)


def _mk_emb_map(s):
    return lambda i: (jnp.minimum(_NS * i + s, _TOTB - 1), 0)


def _tc_dot(emb_table, w2d):
    return pl.pallas_call(
        _tc_dot_body,
        grid=(_TCG,),
        in_specs=[pl.BlockSpec((_BLK, D), _mk_emb_map(s)) for s in range(_NS)]
        + [pl.BlockSpec((D, 1), lambda i: (0, 0))],
        out_specs=pl.BlockSpec((1, _STEP), lambda i: (0, i)),
        out_shape=jax.ShapeDtypeStruct((1, _VPAD), jnp.float32),
    )(*([emb_table] * _NS), w2d)


_TCROWS = 0
_SCROWS = VOCAB - _TCROWS
_WSTRIDE = 31248
_WSPAN = 31312
_CH = 256
_CHG = _CH // 16
_NCH = -(-(_WSPAN - _CH) // _CH) + 1


def _sc_dot_body(emb_hbm, w_hbm, v_hbm, buf0, buf1, w_v, vout_v, sem0, sem1):
    wid = lax.axis_index("s") * 2 + lax.axis_index("c")
    wb = _TCROWS + wid * _WSTRIDE
    pltpu.sync_copy(w_hbm, w_v)

    def chunk_start(c):
        return wb + jnp.minimum(c * _CH, _WSPAN - _CH)

    pltpu.async_copy(emb_hbm.at[pl.ds(chunk_start(0), _CH), :], buf0, sem0)

    wseg = [w_v[pl.ds(m * 16, 16)] for m in range(D // 16)]
    lane = lax.iota(jnp.int32, 16)

    def compute(buf, off):
        def group(g, carry):
            out16 = jnp.zeros((16,), jnp.float32)
            for r in range(16):
                row = g * 16 + r
                sv = buf[row, pl.ds(0, 16)] * wseg[0]
                for m in range(1, D // 16):
                    sv = sv + buf[row, pl.ds(m * 16, 16)] * wseg[m]
                tot16 = sv
                for h in (8, 4, 2, 1):
                    tot16 = tot16 + lax.gather(
                        tot16, (lane ^ h)[:, None],
                        dimension_numbers=lax.GatherDimensionNumbers(
                            offset_dims=(), collapsed_slice_dims=(0,),
                            start_index_map=(0,)),
                        slice_sizes=(1,),
                        mode=lax.GatherScatterMode.PROMISE_IN_BOUNDS)
                out16 = jnp.where(lane == r, tot16, out16)
            vout_v[pl.ds(off + g * 16, 16)] = out16
            return carry

        lax.fori_loop(0, _CHG, group, 0)

    def chunk(c, carry):
        @pl.when(c + 1 < _NCH)
        def _():
            nxt = chunk_start(c + 1)

            @pl.when(c % 2 == 0)
            def _():
                pltpu.async_copy(emb_hbm.at[pl.ds(nxt, _CH), :], buf1, sem1)

            @pl.when(c % 2 == 1)
            def _():
                pltpu.async_copy(emb_hbm.at[pl.ds(nxt, _CH), :], buf0, sem0)

        off = chunk_start(c) - wb

        @pl.when(c % 2 == 0)
        def _():
            pltpu.make_async_copy(
                emb_hbm.at[pl.ds(wb, _CH), :], buf0, sem0).wait()
            compute(buf0, off)

        @pl.when(c % 2 == 1)
        def _():
            pltpu.make_async_copy(
                emb_hbm.at[pl.ds(wb, _CH), :], buf1, sem1).wait()
            compute(buf1, off)

        return carry

    lax.fori_loop(0, _NCH, chunk, 0)
    pltpu.sync_copy(vout_v, v_hbm.at[pl.ds(wid * _WSTRIDE, _WSPAN)])


def _sc_dot(emb_table, w64):
    mesh = plsc.VectorSubcoreMesh(core_axis_name="c", subcore_axis_name="s")
    f = pl.kernel(
        _sc_dot_body,
        mesh=mesh,
        out_type=jax.ShapeDtypeStruct((_SCROWS,), jnp.float32),
        scratch_types=[
            pltpu.VMEM((_CH, D), jnp.float32),
            pltpu.VMEM((_CH, D), jnp.float32),
            pltpu.VMEM((D,), jnp.float32),
            pltpu.VMEM((_WSPAN,), jnp.float32),
            pltpu.SemaphoreType.DMA,
            pltpu.SemaphoreType.DMA,
        ],
    )
    return f(emb_table, w64)


def _sc_body(a_hbm, v_hbm, bias_hbm, out_hbm, idx_v, vals_v, acc_v, bias_v, sem):
    wid = lax.axis_index("s") * 2 + lax.axis_index("c")
    pltpu.sync_copy(a_hbm.at[wid], idx_v)
    pltpu.sync_copy(bias_hbm, bias_v)

    def _fire(j, carry):
        pltpu.async_copy(v_hbm.at[idx_v.at[j]], vals_v.at[j], sem)
        return carry

    lax.fori_loop(0, L, _fire, 0)
    pltpu.make_async_copy(a_hbm.at[wid], idx_v, sem).wait()

    bias = bias_v[...]

    def _acc(j, accs):
        return tuple(
            accs[g] + vals_v[j, pl.ds(g * 16, 16)] for g in range(NGRP)
        )

    accs = lax.fori_loop(
        0, L, _acc, tuple(jnp.zeros((16,), jnp.float32) for _ in range(NGRP))
    )
    for g in range(NGRP):
        acc_v[pl.ds(g * 16, 16)] = accs[g] + bias
    pltpu.sync_copy(acc_v, out_hbm.at[pl.ds(wid * BPW, BPW)])


def _sc_gather_sum(a, v_flat, bias16):
    mesh = plsc.VectorSubcoreMesh(core_axis_name="c", subcore_axis_name="s")
    f = pl.kernel(
        _sc_body,
        mesh=mesh,
        out_type=jax.ShapeDtypeStruct((B,), jnp.float32),
        scratch_types=[
            pltpu.VMEM((L, BPW), jnp.int32),
            pltpu.VMEM((L, BPW), jnp.float32),
            pltpu.VMEM((BPW,), jnp.float32),
            pltpu.VMEM((16,), jnp.float32),
            pltpu.SemaphoreType.DMA,
        ],
    )
    return f(a, v_flat, bias16)


def kernel(inp, emb_table, fc_w, fc_b):
    w2d = fc_w.astype(jnp.float32) / L
    v = _sc_dot(emb_table, w2d.reshape(D))
    a = inp.astype(jnp.int32).reshape(NW, BPW, L).transpose(0, 2, 1)
    bias16 = jnp.broadcast_to(fc_b.astype(jnp.float32), (16,))
    return _sc_gather_sum(a, v, bias16)


# speedup vs baseline: 1.0913x; 1.0913x over previous
import jax
import jax.numpy as jnp
from jax import lax
from jax.experimental import pallas as pl
from jax.experimental.pallas import tpu as pltpu
from jax.experimental.pallas import tpu_sc as plsc

VOCAB = 1000000
D = 64
B = 4096
L = 200
NW = 32
BPW = B // NW
NGRP = BPW // 16


_NS = 4
_BLK = 8192
_STEP = _NS * _BLK
_TCG = -(-VOCAB // _STEP)
_VPAD = _TCG * _STEP


def _tc_dot_body(*refs):
    w_ref, o_ref = refs[_NS], refs[_NS + 1]
    for s in range(_NS):
        y = jnp.dot(refs[s][...], w_ref[...], preferred_element_type=jnp.float32)
        o_ref[:, s * _BLK:(s + 1) * _BLK] = y.T


_TOTB = -(-VOCAB // _BLK)


def _mk_emb_map(s):
    return lambda i: (jnp.minimum(_NS * i + s, _TOTB - 1), 0)


def _tc_dot(emb_table, w2d):
    return pl.pallas_call(
        _tc_dot_body,
        grid=(_TCG,),
        in_specs=[pl.BlockSpec((_BLK, D), _mk_emb_map(s)) for s in range(_NS)]
        + [pl.BlockSpec((D, 1), lambda i: (0, 0))],
        out_specs=pl.BlockSpec((1, _STEP), lambda i: (0, i)),
        out_shape=jax.ShapeDtypeStruct((1, _VPAD), jnp.float32),
    )(*([emb_table] * _NS), w2d)


def _sc_body(a_hbm, v_hbm, bias_hbm, out_hbm, idx_v, vals_v, acc_v, bias_v, sem):
    wid = lax.axis_index("s") * 2 + lax.axis_index("c")
    pltpu.sync_copy(a_hbm.at[wid], idx_v)
    pltpu.sync_copy(bias_hbm, bias_v)

    def _fire(j, carry):
        pltpu.async_copy(v_hbm.at[idx_v.at[j]], vals_v.at[j], sem)
        return carry

    lax.fori_loop(0, L, _fire, 0)
    pltpu.make_async_copy(a_hbm.at[wid], idx_v, sem).wait()

    bias = bias_v[...]

    def _acc(j, accs):
        return tuple(
            accs[g] + vals_v[j, pl.ds(g * 16, 16)] for g in range(NGRP)
        )

    accs = lax.fori_loop(
        0, L, _acc, tuple(jnp.zeros((16,), jnp.float32) for _ in range(NGRP))
    )
    for g in range(NGRP):
        acc_v[pl.ds(g * 16, 16)] = accs[g] + bias
    pltpu.sync_copy(acc_v, out_hbm.at[pl.ds(wid * BPW, BPW)])


def _sc_gather_sum(a, v_flat, bias16):
    mesh = plsc.VectorSubcoreMesh(core_axis_name="c", subcore_axis_name="s")
    f = pl.kernel(
        _sc_body,
        mesh=mesh,
        out_type=jax.ShapeDtypeStruct((B,), jnp.float32),
        scratch_types=[
            pltpu.VMEM((L, BPW), jnp.int32),
            pltpu.VMEM((L, BPW), jnp.float32),
            pltpu.VMEM((BPW,), jnp.float32),
            pltpu.VMEM((16,), jnp.float32),
            pltpu.SemaphoreType.DMA,
        ],
    )
    return f(a, v_flat, bias16)


def kernel(inp, emb_table, fc_w, fc_b):
    w2d = fc_w.astype(jnp.float32) / L
    v2 = _tc_dot(emb_table, w2d)
    a = inp.astype(jnp.int32).reshape(NW, BPW, L).transpose(0, 2, 1)
    bias16 = jnp.broadcast_to(fc_b.astype(jnp.float32), (16,))
    return _sc_gather_sum(a, v2.reshape(_VPAD), bias16)
